# baseline (speedup 1.0000x reference)
import functools

import jax
import jax.numpy as jnp
from jax import lax
from jax.experimental import pallas as pl
from jax.experimental.pallas import tpu as pltpu
from jax.experimental.pallas import tpu_sc as plsc

D_MODEL_H = 768
N_EXP_H = 64
NEG_INF_H = float("-inf")
CHUNKS_H = ((16384, 4096), (16384, 4096))

_INFO = plsc.get_sparse_core_info()
_NW = _INFO.num_cores * _INFO.num_subcores
_L = _INFO.num_lanes


def _matmul_block(w_ref, x_ref, out_ref, *, slabs, toks):
    logits_t = lax.dot_general(
        w_ref[...], x_ref[...], (((1,), (1,)), ((), ())),
        preferred_element_type=jnp.float32)
    for j in range(slabs):
        out_ref[j] = logits_t[:, j * toks:(j + 1) * toks]


def _tc_logits(x_flat, W, start_tok, n_c, T, toks):
    slabs = T // toks
    tiles = n_c // T
    tile0 = start_tok // T
    out = pl.pallas_call(
        functools.partial(_matmul_block, slabs=slabs, toks=toks),
        grid=(tiles,),
        in_specs=[
            pl.BlockSpec((N_EXP_H, D_MODEL_H), lambda i: (0, 0)),
            pl.BlockSpec((T, D_MODEL_H), lambda i, t0=tile0: (i + t0, 0)),
        ],
        out_specs=pl.BlockSpec((slabs, N_EXP_H, toks), lambda i: (i, 0, 0)),
        out_shape=jax.ShapeDtypeStruct(
            (n_c // toks, N_EXP_H, toks), jnp.float32),
    )(W, x_flat)
    return out.reshape(n_c // toks, N_EXP_H * toks)


def _sc_body(logits_hbm, g1_hbm, g2_hbm, i1_hbm, i2_hbm, buf, g1b, g2b,
             i1b, i2b, *, toks):
    wid = lax.axis_index("s") * _INFO.num_cores + lax.axis_index("c")
    pltpu.sync_copy(logits_hbm.at[wid], buf)

    def blk_body(blk, _):
        o = blk * _L
        m1 = buf[pl.ds(o, _L)]
        a1 = jnp.zeros((_L,), jnp.int32)
        m2 = jnp.full((_L,), NEG_INF_H, jnp.float32)
        a2 = jnp.zeros((_L,), jnp.int32)
        for e in range(1, N_EXP_H):
            l = buf[pl.ds(e * toks + o, _L)]
            es = jnp.full((_L,), e, jnp.int32)
            gt1 = l > m1
            gt2 = l > m2
            m2n = jnp.maximum(m2, jnp.minimum(l, m1))
            a2 = jnp.where(gt1, a1, jnp.where(gt2, es, a2))
            m1 = jnp.maximum(m1, l)
            a1 = jnp.where(gt1, es, a1)
            m2 = m2n
        ex = jnp.exp(m2 - m1)
        g1 = 1.0 / (1.0 + ex)
        g1b[pl.ds(o, _L)] = g1
        g2b[pl.ds(o, _L)] = 1.0 - g1
        i1b[pl.ds(o, _L)] = a1
        i2b[pl.ds(o, _L)] = a2
        return ()

    lax.fori_loop(0, toks // _L, blk_body, ())
    base = wid * toks
    pltpu.sync_copy(g1b, g1_hbm.at[pl.ds(base, toks)])
    pltpu.sync_copy(g2b, g2_hbm.at[pl.ds(base, toks)])
    pltpu.sync_copy(i1b, i1_hbm.at[pl.ds(base, toks)])
    pltpu.sync_copy(i2b, i2_hbm.at[pl.ds(base, toks)])


def _sc_top2(logits_slabs, toks):
    n_tok = _NW * toks
    mesh = plsc.VectorSubcoreMesh(core_axis_name="c", subcore_axis_name="s")
    f = pl.kernel(
        functools.partial(_sc_body, toks=toks),
        mesh=mesh,
        out_type=[
            jax.ShapeDtypeStruct((n_tok,), jnp.float32),
            jax.ShapeDtypeStruct((n_tok,), jnp.float32),
            jax.ShapeDtypeStruct((n_tok,), jnp.int32),
            jax.ShapeDtypeStruct((n_tok,), jnp.int32),
        ],
        scratch_types=[
            pltpu.VMEM((N_EXP_H * toks,), jnp.float32),
            pltpu.VMEM((toks,), jnp.float32),
            pltpu.VMEM((toks,), jnp.float32),
            pltpu.VMEM((toks,), jnp.int32),
            pltpu.VMEM((toks,), jnp.int32),
        ],
    )
    return f(logits_slabs)


def _interleave_block(g10, g20, i10, i20, g11, g21, i11, i21,
                      gates_ref, idx_ref, *, rows):
    def pair(a_ref, b_ref):
        return jnp.stack([a_ref[...], b_ref[...]], axis=-1).reshape(
            rows, 256)

    gates_ref[0:rows] = pair(g10, g20)
    gates_ref[rows:2 * rows] = pair(g11, g21)
    idx_ref[0:rows] = pair(i10, i20)
    idx_ref[rows:2 * rows] = pair(i11, i21)


def _assemble(parts, n_tok):
    half = n_tok // 2
    rows = half // 128
    ins = []
    for p in parts:
        ins.extend([p[0].reshape(rows, 128), p[1].reshape(rows, 128),
                    p[2].reshape(rows, 128), p[3].reshape(rows, 128)])
    ins = [ins[0], ins[1], ins[2], ins[3], ins[4], ins[5], ins[6], ins[7]]
    specs = [pl.BlockSpec((rows, 128), lambda i: (0, 0)) for _ in ins]
    gates, idx = pl.pallas_call(
        functools.partial(_interleave_block, rows=rows),
        grid=(1,),
        in_specs=specs,
        out_specs=[
            pl.BlockSpec((2 * rows, 256), lambda i: (0, 0)),
            pl.BlockSpec((2 * rows, 256), lambda i: (0, 0)),
        ],
        out_shape=[
            jax.ShapeDtypeStruct((2 * rows, 256), jnp.float32),
            jax.ShapeDtypeStruct((2 * rows, 256), jnp.int32),
        ],
    )(*ins)
    return gates, idx


def kernel(x, W):
    B, S, D = x.shape
    n_tok = B * S
    xf = x.reshape(n_tok, D)
    parts = []
    start = 0
    for n_c, T in CHUNKS_H:
        toks = n_c // _NW
        parts.append(_sc_top2(_tc_logits(xf, W, start, n_c, T, toks), toks))
        start += n_c
    gates, idx = _assemble(parts, n_tok)
    return gates.reshape(B, S, 2), idx.reshape(B, S, 2)


# speedup vs baseline: 1.9404x; 1.9404x over previous
import functools

import jax
import jax.numpy as jnp
from jax import lax
from jax.experimental import pallas as pl
from jax.experimental.pallas import tpu as pltpu
from jax.experimental.pallas import tpu_sc as plsc

D_MODEL_H = 768
N_EXP_H = 64
NEG_INF_H = float("-inf")
CHUNKS_H = ((24576, 3072), (8192, 4096))

_INFO = plsc.get_sparse_core_info()
_NW = _INFO.num_cores * _INFO.num_subcores
_L = _INFO.num_lanes


def _matmul_block(w_ref, x_ref, out_ref, *, slabs, toks):
    logits_t = lax.dot_general(
        w_ref[...], x_ref[...], (((1,), (1,)), ((), ())),
        preferred_element_type=jnp.float32)
    for j in range(slabs):
        out_ref[j] = logits_t[:, j * toks:(j + 1) * toks]


def _tc_logits(x_flat, W, start_tok, n_c, T, toks):
    slabs = T // toks
    tiles = n_c // T
    tile0 = start_tok // T
    out = pl.pallas_call(
        functools.partial(_matmul_block, slabs=slabs, toks=toks),
        grid=(tiles,),
        in_specs=[
            pl.BlockSpec((N_EXP_H, D_MODEL_H), lambda i: (0, 0)),
            pl.BlockSpec((T, D_MODEL_H), lambda i, t0=tile0: (i + t0, 0)),
        ],
        out_specs=pl.BlockSpec((slabs, N_EXP_H, toks), lambda i: (i, 0, 0)),
        out_shape=jax.ShapeDtypeStruct(
            (n_c // toks, N_EXP_H, toks), jnp.float32),
    )(W, x_flat)
    return out.reshape(n_c // toks, N_EXP_H * toks)


def _sc_body(logits_hbm, g1_hbm, g2_hbm, i1_hbm, i2_hbm, buf, g1b, g2b,
             i1b, i2b, *, toks):
    wid = lax.axis_index("s") * _INFO.num_cores + lax.axis_index("c")
    pltpu.sync_copy(logits_hbm.at[wid], buf)

    def blk_body(blk, _):
        o = blk * _L
        m1 = buf[pl.ds(o, _L)]
        a1 = jnp.zeros((_L,), jnp.int32)
        m2 = jnp.full((_L,), NEG_INF_H, jnp.float32)
        a2 = jnp.zeros((_L,), jnp.int32)
        for e in range(1, N_EXP_H):
            l = buf[pl.ds(e * toks + o, _L)]
            es = jnp.full((_L,), e, jnp.int32)
            gt1 = l > m1
            gt2 = l > m2
            m2n = jnp.maximum(m2, jnp.minimum(l, m1))
            a2 = jnp.where(gt1, a1, jnp.where(gt2, es, a2))
            m1 = jnp.maximum(m1, l)
            a1 = jnp.where(gt1, es, a1)
            m2 = m2n
        ex = jnp.exp(m2 - m1)
        g1 = 1.0 / (1.0 + ex)
        g1b[pl.ds(o, _L)] = g1
        g2b[pl.ds(o, _L)] = 1.0 - g1
        i1b[pl.ds(o, _L)] = a1
        i2b[pl.ds(o, _L)] = a2
        return ()

    lax.fori_loop(0, toks // _L, blk_body, ())
    base = wid * toks
    pltpu.sync_copy(g1b, g1_hbm.at[pl.ds(base, toks)])
    pltpu.sync_copy(g2b, g2_hbm.at[pl.ds(base, toks)])
    pltpu.sync_copy(i1b, i1_hbm.at[pl.ds(base, toks)])
    pltpu.sync_copy(i2b, i2_hbm.at[pl.ds(base, toks)])


def _sc_top2(logits_slabs, toks):
    n_tok = _NW * toks
    mesh = plsc.VectorSubcoreMesh(core_axis_name="c", subcore_axis_name="s")
    f = pl.kernel(
        functools.partial(_sc_body, toks=toks),
        mesh=mesh,
        out_type=[
            jax.ShapeDtypeStruct((n_tok,), jnp.float32),
            jax.ShapeDtypeStruct((n_tok,), jnp.float32),
            jax.ShapeDtypeStruct((n_tok,), jnp.int32),
            jax.ShapeDtypeStruct((n_tok,), jnp.int32),
        ],
        scratch_types=[
            pltpu.VMEM((N_EXP_H * toks,), jnp.float32),
            pltpu.VMEM((toks,), jnp.float32),
            pltpu.VMEM((toks,), jnp.float32),
            pltpu.VMEM((toks,), jnp.int32),
            pltpu.VMEM((toks,), jnp.int32),
        ],
    )
    return f(logits_slabs)


def kernel(x, W):
    B, S, D = x.shape
    n_tok = B * S
    xf = x.reshape(n_tok, D)
    parts = []
    start = 0
    for n_c, T in CHUNKS_H:
        toks = n_c // _NW
        parts.append(_sc_top2(_tc_logits(xf, W, start, n_c, T, toks), toks))
        start += n_c
    g1 = jnp.concatenate([p[0] for p in parts])
    g2 = jnp.concatenate([p[1] for p in parts])
    i1 = jnp.concatenate([p[2] for p in parts])
    i2 = jnp.concatenate([p[3] for p in parts])
    gates = jnp.stack([g1, g2], axis=-1).reshape(B, S, 2)
    idx = jnp.stack([i1, i2], axis=-1).reshape(B, S, 2)
    return gates, idx


# speedup vs baseline: 2.0242x; 1.0432x over previous
import functools

import jax
import jax.numpy as jnp
from jax import lax
from jax.experimental import pallas as pl
from jax.experimental.pallas import tpu as pltpu
from jax.experimental.pallas import tpu_sc as plsc

D_MODEL_H = 768
N_EXP_H = 64
NEG_INF_H = float("-inf")
CHUNKS_H = ((16384, 4096), (16384, 4096))

_INFO = plsc.get_sparse_core_info()
_NW = _INFO.num_cores * _INFO.num_subcores
_L = _INFO.num_lanes


def _matmul_block(w_ref, x_ref, out_ref, *, slabs, toks):
    logits_t = lax.dot_general(
        w_ref[...], x_ref[...], (((1,), (1,)), ((), ())),
        preferred_element_type=jnp.float32)
    for j in range(slabs):
        out_ref[j] = logits_t[:, j * toks:(j + 1) * toks]


def _tc_logits(x_flat, W, start_tok, n_c, T, toks):
    slabs = T // toks
    tiles = n_c // T
    tile0 = start_tok // T
    out = pl.pallas_call(
        functools.partial(_matmul_block, slabs=slabs, toks=toks),
        grid=(tiles,),
        in_specs=[
            pl.BlockSpec((N_EXP_H, D_MODEL_H), lambda i: (0, 0)),
            pl.BlockSpec((T, D_MODEL_H), lambda i, t0=tile0: (i + t0, 0)),
        ],
        out_specs=pl.BlockSpec((slabs, N_EXP_H, toks), lambda i: (i, 0, 0)),
        out_shape=jax.ShapeDtypeStruct(
            (n_c // toks, N_EXP_H, toks), jnp.float32),
    )(W, x_flat)
    return out.reshape(n_c // toks, N_EXP_H * toks)


def _sc_body(logits_hbm, g1_hbm, g2_hbm, i1_hbm, i2_hbm, buf, g1b, g2b,
             i1b, i2b, *, toks):
    wid = lax.axis_index("s") * _INFO.num_cores + lax.axis_index("c")
    pltpu.sync_copy(logits_hbm.at[wid], buf)

    def blk_body(blk, _):
        o = blk * _L
        m1 = buf[pl.ds(o, _L)]
        a1 = jnp.zeros((_L,), jnp.int32)
        m2 = jnp.full((_L,), NEG_INF_H, jnp.float32)
        a2 = jnp.zeros((_L,), jnp.int32)
        for e in range(1, N_EXP_H):
            l = buf[pl.ds(e * toks + o, _L)]
            es = jnp.full((_L,), e, jnp.int32)
            gt1 = l > m1
            gt2 = l > m2
            m2n = jnp.maximum(m2, jnp.minimum(l, m1))
            a2 = jnp.where(gt1, a1, jnp.where(gt2, es, a2))
            m1 = jnp.maximum(m1, l)
            a1 = jnp.where(gt1, es, a1)
            m2 = m2n
        ex = jnp.exp(m2 - m1)
        g1 = 1.0 / (1.0 + ex)
        g1b[pl.ds(o, _L)] = g1
        g2b[pl.ds(o, _L)] = 1.0 - g1
        i1b[pl.ds(o, _L)] = a1
        i2b[pl.ds(o, _L)] = a2
        return ()

    lax.fori_loop(0, toks // _L, blk_body, ())
    base = wid * toks
    pltpu.sync_copy(g1b, g1_hbm.at[pl.ds(base, toks)])
    pltpu.sync_copy(g2b, g2_hbm.at[pl.ds(base, toks)])
    pltpu.sync_copy(i1b, i1_hbm.at[pl.ds(base, toks)])
    pltpu.sync_copy(i2b, i2_hbm.at[pl.ds(base, toks)])


def _sc_top2(logits_slabs, toks):
    n_tok = _NW * toks
    mesh = plsc.VectorSubcoreMesh(core_axis_name="c", subcore_axis_name="s")
    f = pl.kernel(
        functools.partial(_sc_body, toks=toks),
        mesh=mesh,
        out_type=[
            jax.ShapeDtypeStruct((n_tok,), jnp.float32),
            jax.ShapeDtypeStruct((n_tok,), jnp.float32),
            jax.ShapeDtypeStruct((n_tok,), jnp.int32),
            jax.ShapeDtypeStruct((n_tok,), jnp.int32),
        ],
        scratch_types=[
            pltpu.VMEM((N_EXP_H * toks,), jnp.float32),
            pltpu.VMEM((toks,), jnp.float32),
            pltpu.VMEM((toks,), jnp.float32),
            pltpu.VMEM((toks,), jnp.int32),
            pltpu.VMEM((toks,), jnp.int32),
        ],
    )
    return f(logits_slabs)


def kernel(x, W):
    B, S, D = x.shape
    n_tok = B * S
    xf = x.reshape(n_tok, D)
    parts = []
    start = 0
    for n_c, T in CHUNKS_H:
        toks = n_c // _NW
        parts.append(_sc_top2(_tc_logits(xf, W, start, n_c, T, toks), toks))
        start += n_c
    g1 = jnp.concatenate([p[0] for p in parts])
    g2 = jnp.concatenate([p[1] for p in parts])
    i1 = jnp.concatenate([p[2] for p in parts])
    i2 = jnp.concatenate([p[3] for p in parts])
    gates = jnp.stack([g1, g2], axis=-1).reshape(B, S, 2)
    idx = jnp.stack([i1, i2], axis=-1).reshape(B, S, 2)
    return gates, idx
